# baseline (speedup 1.0000x reference)
import functools

import jax
import jax.numpy as jnp
from jax import lax
from jax.experimental import pallas as pl
from jax.experimental.pallas import tpu as pltpu
from jax.experimental.pallas import tpu_sc as plsc

N_EMB = 32
CHUNK = 128


def _gmf_kernel(b_per_w, users_hbm, items_hbm, utab_hbm, itab_hbm,
                w_hbm, b_hbm, out_hbm,
                idx_u_v, idx_i_v, rows_u_v, rows_i_v, w_v, b_v, out_v, sem):
    n_chunks = b_per_w // CHUNK
    wid = lax.axis_index("s") * 2 + lax.axis_index("c")
    base_row = wid * n_chunks

    pltpu.sync_copy(users_hbm.at[pl.ds(base_row, n_chunks)], idx_u_v)
    pltpu.sync_copy(items_hbm.at[pl.ds(base_row, n_chunks)], idx_i_v)
    pltpu.sync_copy(w_hbm, w_v)
    pltpu.sync_copy(b_hbm, b_v)

    copies = []
    for j in range(n_chunks):
        copies.append(pltpu.async_copy(
            utab_hbm.at[idx_u_v.at[j]],
            rows_u_v.at[pl.ds(j * CHUNK, CHUNK)], sem))
        copies.append(pltpu.async_copy(
            itab_hbm.at[idx_i_v.at[j]],
            rows_i_v.at[pl.ds(j * CHUNK, CHUNK)], sem))
    for c in copies:
        c.wait()

    lane = lax.iota(jnp.int32, 16)
    e_ids = [jnp.full((16,), e, jnp.int32) for e in range(N_EMB)]
    w_cols = [w_v[e, :] for e in range(N_EMB)]
    b_vec = b_v[...]

    def group_body(g, _):
        row = g * 16 + lane
        acc = jnp.zeros((16,), jnp.float32)
        for e in range(N_EMB):
            u_col = plsc.load_gather(rows_u_v, [row, e_ids[e]])
            i_col = plsc.load_gather(rows_i_v, [row, e_ids[e]])
            acc = acc + u_col * i_col * w_cols[e]
        logit = acc + b_vec
        out_v[pl.ds(g * 16, 16)] = 1.0 / (1.0 + jnp.exp(-logit))
        return 0

    lax.fori_loop(0, b_per_w // 16, group_body, 0)

    pltpu.sync_copy(out_v, out_hbm.at[pl.ds(wid * b_per_w, b_per_w)])


def kernel(users, items, user_table, item_table, W, b):
    batch = users.shape[0]
    info = plsc.get_sparse_core_info()
    nw = info.num_cores * info.num_subcores
    b_per_w = batch // nw
    n_chunks = b_per_w // CHUNK

    users2 = users.reshape(batch // CHUNK, CHUNK)
    items2 = items.reshape(batch // CHUNK, CHUNK)
    w_bcast = jnp.broadcast_to(W.reshape(N_EMB, 1), (N_EMB, 16))
    b16 = jnp.broadcast_to(b, (16,))

    mesh = plsc.VectorSubcoreMesh(core_axis_name="c", subcore_axis_name="s")
    run = functools.partial(
        pl.kernel,
        mesh=mesh,
        out_type=jax.ShapeDtypeStruct((batch,), jnp.float32),
        compiler_params=pltpu.CompilerParams(
            needs_layout_passes=False, use_tc_tiling_on_sc=False),
        scratch_types=[
            pltpu.VMEM((n_chunks, CHUNK), jnp.int32),
            pltpu.VMEM((n_chunks, CHUNK), jnp.int32),
            pltpu.VMEM((b_per_w, N_EMB), jnp.float32),
            pltpu.VMEM((b_per_w, N_EMB), jnp.float32),
            pltpu.VMEM((N_EMB, 16), jnp.float32),
            pltpu.VMEM((16,), jnp.float32),
            pltpu.VMEM((b_per_w,), jnp.float32),
            pltpu.SemaphoreType.DMA,
        ],
    )(functools.partial(_gmf_kernel, b_per_w))

    out = run(users2, items2, user_table, item_table, w_bcast, b16)
    return out.reshape(batch, 1)


# speedup vs baseline: 3.7083x; 3.7083x over previous
import functools

import jax
import jax.numpy as jnp
from jax import lax
from jax.experimental import pallas as pl
from jax.experimental.pallas import tpu as pltpu
from jax.experimental.pallas import tpu_sc as plsc

N_EMB = 32
GRP = 16


def _gmf_kernel(b_per_w, users_hbm, items_hbm, utabT_hbm, itabT_hbm,
                w_hbm, b_hbm, out_hbm,
                idx_u_v, idx_i_v, ring, packed_u, w_v, b_v, out_v, sem):
    wid = lax.axis_index("s") * 2 + lax.axis_index("c")
    base = wid * b_per_w
    n_grp = b_per_w // GRP

    pltpu.sync_copy(users_hbm.at[pl.ds(base, b_per_w)],
                    idx_u_v.at[pl.ds(0, b_per_w)])
    pltpu.sync_copy(items_hbm.at[pl.ds(base, b_per_w)],
                    idx_i_v.at[pl.ds(0, b_per_w)])
    pltpu.sync_copy(w_hbm, w_v)
    pltpu.sync_copy(b_hbm, b_v)

    lane = lax.iota(jnp.int32, 16)
    e_ids = [jnp.full((16,), e, jnp.int32) for e in range(N_EMB)]
    w_cols = [w_v[e, :] for e in range(N_EMB)]
    b_vec = b_v[...]

    def fetch_blocks(g, idx_ref, tab_hbm):
        copies = []
        for jj in range(GRP):
            r = idx_ref[pl.ds(g * GRP + jj, 16)][0]
            c0 = pl.multiple_of((r >> 7) << 7, 128)
            copies.append(pltpu.async_copy(
                tab_hbm.at[:, pl.ds(c0, 128)], ring.at[jj], sem))
        for c in copies:
            c.wait()

    def user_body(g, _):
        fetch_blocks(g, idx_u_v, utabT_hbm)
        o_vec = idx_u_v[pl.ds(g * GRP, 16)] & 127
        for e in range(N_EMB):
            col = plsc.load_gather(ring, [lane, e_ids[e], o_vec])
            packed_u[e, pl.ds(g * GRP, 16)] = col
        return 0

    lax.fori_loop(0, n_grp, user_body, 0)

    def item_body(g, _):
        fetch_blocks(g, idx_i_v, itabT_hbm)
        o_vec = idx_i_v[pl.ds(g * GRP, 16)] & 127
        acc = jnp.zeros((16,), jnp.float32)
        for e in range(N_EMB):
            i_col = plsc.load_gather(ring, [lane, e_ids[e], o_vec])
            acc = acc + packed_u[e, pl.ds(g * GRP, 16)] * i_col * w_cols[e]
        logit = acc + b_vec
        out_v[pl.ds(g * GRP, 16)] = 1.0 / (1.0 + jnp.exp(-logit))
        return 0

    lax.fori_loop(0, n_grp, item_body, 0)

    pltpu.sync_copy(out_v, out_hbm.at[pl.ds(base, b_per_w)])


def kernel(users, items, user_table, item_table, W, b):
    batch = users.shape[0]
    info = plsc.get_sparse_core_info()
    nw = info.num_cores * info.num_subcores
    b_per_w = batch // nw

    utabT = user_table.T
    itabT = item_table.T
    w_bcast = jnp.broadcast_to(W.reshape(N_EMB, 1), (N_EMB, 16))
    b16 = jnp.broadcast_to(b, (16,))

    mesh = plsc.VectorSubcoreMesh(core_axis_name="c", subcore_axis_name="s")
    run = functools.partial(
        pl.kernel,
        mesh=mesh,
        out_type=jax.ShapeDtypeStruct((batch,), jnp.float32),
        compiler_params=pltpu.CompilerParams(
            needs_layout_passes=False, use_tc_tiling_on_sc=True),
        scratch_types=[
            pltpu.VMEM((b_per_w + 16,), jnp.int32),
            pltpu.VMEM((b_per_w + 16,), jnp.int32),
            pltpu.VMEM((GRP, N_EMB, 128), jnp.float32),
            pltpu.VMEM((N_EMB, b_per_w), jnp.float32),
            pltpu.VMEM((N_EMB, 16), jnp.float32),
            pltpu.VMEM((16,), jnp.float32),
            pltpu.VMEM((b_per_w,), jnp.float32),
            pltpu.SemaphoreType.DMA,
        ],
    )(functools.partial(_gmf_kernel, b_per_w))

    out = run(users, items, utabT, itabT, w_bcast, b16)
    return out.reshape(batch, 1)
